# Initial kernel scaffold; baseline (speedup 1.0000x reference)
import jax
import jax.numpy as jnp
from jax.experimental import pallas as pl


def kernel(tensor, values, indices):
    raise NotImplementedError("write your pallas kernel here")


# speedup vs baseline: 9.9372x; 9.9372x over previous
import functools

import jax
import jax.numpy as jnp
from jax import lax
from jax.experimental import pallas as pl
from jax.experimental.pallas import tpu as pltpu
from jax.experimental.pallas import tpu_sc as plsc

_SHAPE = (4096, 4096)
_FLAT = _SHAPE[0] * _SHAPE[1]
_K = 1048576
_NC, _NS = 2, 16
_NW = _NC * _NS
_BLK = 65536
_NBLK = _FLAT // _BLK
_BPW = _NBLK // _NW
_Q = 2048
_G = _Q // 16

_mesh = plsc.VectorSubcoreMesh(core_axis_name="c", subcore_axis_name="s")


@functools.partial(
    pl.kernel,
    out_type=jax.ShapeDtypeStruct((_FLAT,), jnp.float32),
    mesh=_mesh,
    compiler_params=pltpu.CompilerParams(needs_layout_passes=False),
    scratch_types=[
        pltpu.VMEM((_BLK,), jnp.float32),
        pltpu.VMEM((_Q,), jnp.int32),
        pltpu.VMEM((_Q,), jnp.float32),
        pltpu.VMEM((16,), jnp.int32),
    ],
)
def _sc_scatter_add(tensor_hbm, values_hbm, indices_hbm, bounds_hbm, out_hbm,
                    blk_v, idx_v, val_v, bnd_v):
    wid = lax.axis_index("s") * _NC + lax.axis_index("c")
    pltpu.sync_copy(bounds_hbm.at[pl.ds(wid * _BPW, 16)], bnd_v)
    bv = bnd_v[...]

    for b in range(_BPW):
        g = wid * _BPW + b
        blk_lo = g * _BLK
        p0 = bv[b]
        p1 = bv[b + 1]

        pltpu.sync_copy(tensor_hbm.at[pl.ds(blk_lo, _BLK)], blk_v)

        r0 = p0 // _Q
        r1 = (p1 + _Q - 1) // _Q

        def chunk_body(r, carry, blk_lo=blk_lo):
            base = r * _Q
            pltpu.sync_copy(indices_hbm.at[pl.ds(base, _Q)], idx_v)
            pltpu.sync_copy(values_hbm.at[pl.ds(base, _Q)], val_v)

            def grp(gi, c2, blk_lo=blk_lo):
                iv = idx_v[pl.ds(gi * 16, 16)]
                vv = val_v[pl.ds(gi * 16, 16)]
                m = (iv >= blk_lo) & (iv < blk_lo + _BLK)
                liv = jnp.where(m, iv - blk_lo, 0)
                plsc.addupdate_scatter(blk_v, [liv], vv, mask=m)
                return c2

            lax.fori_loop(0, _G, grp, 0)
            return carry

        lax.fori_loop(r0, r1, chunk_body, 0)
        pltpu.sync_copy(blk_v, out_hbm.at[pl.ds(blk_lo, _BLK)])


def kernel(tensor, values, indices):
    flat = tensor.reshape(-1)
    queries = jnp.arange(_NBLK + 1, dtype=jnp.int32) * _BLK
    bounds = jnp.searchsorted(indices, queries, side="left").astype(jnp.int32)
    bounds = jnp.concatenate([bounds, jnp.full((16,), _K, jnp.int32)])
    out = _sc_scatter_add(flat, values, indices, bounds)
    return out.reshape(_SHAPE)
